# baseline (speedup 1.0000x reference)
import functools

import jax
import jax.numpy as jnp
from jax import lax
from jax.experimental import pallas as pl
from jax.experimental.pallas import tpu as pltpu
from jax.experimental.pallas import tpu_sc as plsc

B = 8
M = 65536
D = 64
SCALE = 2.5
SLOPE = 0.01

MSC = 0
MS = M - MSC

SEG = 4
NW = 32
KEYS = max(MSC // SEG, 1024)
CH = 256
NCH = KEYS // CH
GR = CH // 16

CHT = 4096
NCT = MS // CHT

_LOG2E = 1.4426950408889634
_LN2 = 0.6931471805599453
_RND = 12582912.0


def _bf16_rne(x):
    i = lax.bitcast_convert_type(x, jnp.int32)
    lsb = lax.shift_right_logical(i, 16) & 1
    r = (i + 0x7FFF + lsb) & jnp.int32(-65536)
    return lax.bitcast_convert_type(r, jnp.float32)


def _exp_precise(x):
    x = jnp.maximum(x, -87.0)
    t = x * _LOG2E
    n_f = (t + _RND) - _RND
    r = (t - n_f) * _LN2
    p = 1.0 + r * (1.0 + r * (0.5 + r * (1.0 / 6.0 + r * (
        1.0 / 24.0 + r * (1.0 / 120.0 + r * (1.0 / 720.0))))))
    n_i = n_f.astype(jnp.int32)
    scale = lax.bitcast_convert_type((n_i + 127) << 23, jnp.float32)
    return p * scale


_mesh = plsc.VectorSubcoreMesh(core_axis_name="c", subcore_axis_name="s")


@functools.partial(
    pl.kernel,
    mesh=_mesh,
    compiler_params=pltpu.CompilerParams(needs_layout_passes=False),
    out_type=[
        jax.ShapeDtypeStruct((NW * 64,), jnp.float32),
        jax.ShapeDtypeStruct((NW * 32,), jnp.float32),
    ],
    scratch_types=[
        pltpu.VMEM((CH, D), jnp.float32),
        pltpu.VMEM((CH, D), jnp.float32),
        pltpu.VMEM((KEYS,), jnp.float32),
        pltpu.VMEM((KEYS,), jnp.int32),
        pltpu.VMEM((64,), jnp.float32),
        pltpu.VMEM((16,), jnp.float32),
        pltpu.VMEM((64,), jnp.float32),
        pltpu.VMEM((32,), jnp.float32),
        pltpu.SemaphoreType.DMA,
        pltpu.SemaphoreType.DMA,
    ],
)
def _sc_attn(q_hbm, k_hbm, v_hbm, pos_hbm, qp_hbm, acc_out, st_out,
             buf0, buf1, scores, posb, qv, qpb, stacc, ststat, sem0, sem1):
    cid = lax.axis_index("c")
    sid = lax.axis_index("s")
    wid = cid * 16 + sid
    b = wid // SEG
    seg = wid % SEG
    base = MS + seg * KEYS

    pltpu.sync_copy(q_hbm.at[b], qv)
    pltpu.sync_copy(qp_hbm, qpb)
    pltpu.sync_copy(pos_hbm.at[b, pl.ds(base, KEYS)], posb)

    lane = lax.iota(jnp.int32, 16)
    q0 = _bf16_rne(qv[pl.ds(0, 16)])
    q1 = _bf16_rne(qv[pl.ds(16, 16)])
    q2 = _bf16_rne(qv[pl.ds(32, 16)])
    q3 = _bf16_rne(qv[pl.ds(48, 16)])
    qp_vec = qpb[...]

    def kcopy(c, buf, sem):
        return pltpu.make_async_copy(
            k_hbm.at[b, pl.ds(base + c * CH, CH), :], buf, sem)

    def vcopy(c, buf, sem):
        return pltpu.make_async_copy(
            v_hbm.at[b, pl.ds(base + c * CH, CH), :], buf, sem)

    def p1_chunk(c, buf, m_run):
        def group(gi, m_run):
            grow = gi * 16
            svec = jnp.zeros((16,), jnp.float32)
            for j in range(16):
                row = grow + j
                p = (q0 * _bf16_rne(buf[row, pl.ds(0, 16)])
                     + q1 * _bf16_rne(buf[row, pl.ds(16, 16)])
                     + q2 * _bf16_rne(buf[row, pl.ds(32, 16)])
                     + q3 * _bf16_rne(buf[row, pl.ds(48, 16)]))
                svec = jnp.where(lane == j, jnp.sum(p), svec)
            gk = c * GR + gi
            pos16 = posb[pl.ds(gk * 16, 16)].astype(jnp.float32)
            svec = svec * SCALE - SLOPE * jnp.abs(qp_vec - pos16)
            scores[pl.ds(gk * 16, 16)] = svec
            return jnp.maximum(m_run, svec)
        return lax.fori_loop(0, GR, group, m_run)

    kcopy(0, buf0, sem0).start()

    def outer1(i, m_run):
        c0 = 2 * i
        kcopy(c0, buf0, sem0).wait()
        kcopy(c0 + 1, buf1, sem1).start()
        m_run = p1_chunk(c0, buf0, m_run)
        kcopy(c0 + 1, buf1, sem1).wait()

        @pl.when(i < NCH // 2 - 1)
        def _():
            kcopy(c0 + 2, buf0, sem0).start()

        m_run = p1_chunk(c0 + 1, buf1, m_run)
        return m_run

    m_run = lax.fori_loop(0, NCH // 2, outer1,
                          jnp.full((16,), -3e38, jnp.float32))
    gmax = jnp.max(m_run)

    def p2_chunk(c, buf, carry):
        def group(gi, carry):
            a0, a1, a2, a3, ws = carry
            gk = c * GR + gi
            svec = scores[pl.ds(gk * 16, 16)]
            w = _exp_precise(svec - gmax)
            ws = ws + w
            grow = gi * 16
            for j in range(16):
                row = grow + j
                wj = w[j]
                a0 = a0 + wj * buf[row, pl.ds(0, 16)]
                a1 = a1 + wj * buf[row, pl.ds(16, 16)]
                a2 = a2 + wj * buf[row, pl.ds(32, 16)]
                a3 = a3 + wj * buf[row, pl.ds(48, 16)]
            return (a0, a1, a2, a3, ws)
        return lax.fori_loop(0, GR, group, carry)

    vcopy(0, buf0, sem0).start()
    zero = jnp.zeros((16,), jnp.float32)

    def outer2(i, carry):
        c0 = 2 * i
        vcopy(c0, buf0, sem0).wait()
        vcopy(c0 + 1, buf1, sem1).start()
        carry = p2_chunk(c0, buf0, carry)
        vcopy(c0 + 1, buf1, sem1).wait()

        @pl.when(i < NCH // 2 - 1)
        def _():
            vcopy(c0 + 2, buf0, sem0).start()

        carry = p2_chunk(c0 + 1, buf1, carry)
        return carry

    a0, a1, a2, a3, ws = lax.fori_loop(0, NCH // 2, outer2,
                                       (zero, zero, zero, zero, zero))

    stacc[pl.ds(0, 16)] = a0
    stacc[pl.ds(16, 16)] = a1
    stacc[pl.ds(32, 16)] = a2
    stacc[pl.ds(48, 16)] = a3
    pltpu.sync_copy(stacc, acc_out.at[pl.ds(wid * 64, 64)])

    ststat[pl.ds(0, 16)] = zero + gmax
    ststat[pl.ds(16, 16)] = ws
    pltpu.sync_copy(ststat, st_out.at[pl.ds(wid * 32, 32)])


def _tc_body(q_ref, k_hbm, v_hbm, pos_ref, qp_ref, acc_out, st_out,
             kbuf, vbuf, ksems, vsems):
    b = pl.program_id(0)
    c = pl.program_id(1)
    nct = pl.num_programs(1)
    g = b * nct + c

    def dma(hbm, bb, cc, buf, slot, sems):
        return pltpu.make_async_copy(
            hbm.at[bb, pl.ds(cc * CHT, CHT), :], buf.at[slot], sems.at[slot])

    @pl.when(g == 0)
    def _():
        dma(k_hbm, b, c, kbuf, 0, ksems).start()
        dma(v_hbm, b, c, vbuf, 0, vsems).start()

    nb = (g + 1) // nct
    nc = (g + 1) % nct
    for slot in (0, 1):
        @pl.when(jnp.logical_and(g + 1 < B * nct, (g + 1) % 2 == slot))
        def _(slot=slot):
            dma(k_hbm, nb, nc, kbuf, slot, ksems).start()
            dma(v_hbm, nb, nc, vbuf, slot, vsems).start()

    qp = qp_ref[0]
    q16 = q_ref[0].astype(jnp.bfloat16)
    pos = pos_ref[0].astype(jnp.float32)

    def compute(slot):
        dma(k_hbm, b, c, kbuf, slot, ksems).wait()
        dma(v_hbm, b, c, vbuf, slot, vsems).wait()
        k16 = kbuf[slot].astype(jnp.bfloat16)
        s = lax.dot_general(q16, k16, (((1,), (1,)), ((), ())),
                            preferred_element_type=jnp.float32)
        s = s * SCALE - SLOPE * jnp.abs(qp - pos)
        m_c = jnp.max(s)
        p = jnp.exp(s - m_c)
        l_c = jnp.sum(p)
        pv = lax.dot_general(p.astype(jnp.bfloat16),
                             vbuf[slot].astype(jnp.bfloat16),
                             (((1,), (0,)), ((), ())),
                             preferred_element_type=jnp.float32)
        acc_out[...] = pv.reshape(1, 1, 1, 64)
        iota = lax.broadcasted_iota(jnp.int32, (1, 1, 1, 128), 3)
        st_out[...] = jnp.where(iota == 0, m_c,
                                jnp.where(iota == 1, l_c, 0.0))

    for slot in (0, 1):
        @pl.when(g % 2 == slot)
        def _(slot=slot):
            compute(slot)


_tc_attn = pl.pallas_call(
    _tc_body,
    grid=(B, NCT if NCT else 1),
    in_specs=[
        pl.BlockSpec((1, 1, 64), lambda b, c: (b, 0, 0)),
        pl.BlockSpec(memory_space=pl.ANY),
        pl.BlockSpec(memory_space=pl.ANY),
        pl.BlockSpec((1, 1, CHT), lambda b, c: (b * (NCT if NCT else 1) + c, 0, 0)),
        pl.BlockSpec(memory_space=pltpu.MemorySpace.SMEM),
    ],
    out_specs=[
        pl.BlockSpec((1, 1, 1, 64), lambda b, c: (b, c, 0, 0)),
        pl.BlockSpec((1, 1, 1, 128), lambda b, c: (b, c, 0, 0)),
    ],
    out_shape=[
        jax.ShapeDtypeStruct((B, NCT if NCT else 1, 1, 64), jnp.float32),
        jax.ShapeDtypeStruct((B, NCT if NCT else 1, 1, 128), jnp.float32),
    ],
    scratch_shapes=[
        pltpu.VMEM((2, CHT, 64), jnp.float32),
        pltpu.VMEM((2, CHT, 64), jnp.float32),
        pltpu.SemaphoreType.DMA((2,)),
        pltpu.SemaphoreType.DMA((2,)),
    ],
)


def kernel(query_addr, key_addrs, values, query_pos, key_pos):
    pos32 = key_pos.astype(jnp.int32)
    qp_s = jnp.full((1,), query_pos, dtype=jnp.float32)

    parts_m = []
    parts_l = []
    parts_acc = []

    if MSC:
        qp = jnp.full((16,), query_pos, dtype=jnp.float32)
        accf, stf = _sc_attn(query_addr, key_addrs, values, pos32, qp)
        acc = accf.reshape(B, SEG, 64)
        st = stf.reshape(B, SEG, 32)
        parts_m.append(st[:, :, 0])
        parts_l.append(st[:, :, 16:32].sum(-1))
        parts_acc.append(acc)

    if NCT:
        q3d = query_addr.reshape(B, 1, 64)
        pos_tc = pos32[:, :MS].reshape(B * NCT, 1, CHT)
        tacc, tst = _tc_attn(q3d, key_addrs, values, pos_tc, qp_s)
        parts_m.append(tst[:, :, 0, 0])
        parts_l.append(tst[:, :, 0, 1])
        parts_acc.append(tacc[:, :, 0, :])

    pm = jnp.concatenate(parts_m, axis=1)
    pl_ = jnp.concatenate(parts_l, axis=1)
    pa = jnp.concatenate(parts_acc, axis=1)
    gm = pm.max(axis=1, keepdims=True)
    sc = jnp.exp(pm - gm)
    num = (sc[:, :, None] * pa).sum(1)
    den = (sc * pl_).sum(1)[:, None]
    return num / den


# speedup vs baseline: 3.8876x; 3.8876x over previous
import functools

import jax
import jax.numpy as jnp
from jax import lax
from jax.experimental import pallas as pl
from jax.experimental.pallas import tpu as pltpu
from jax.experimental.pallas import tpu_sc as plsc

B = 8
M = 65536
D = 64
SCALE = 2.5
SLOPE = 0.01

MSC = 0
MS = M - MSC

SEG = 4
NW = 32
KEYS = max(MSC // SEG, 1024)
CH = 256
NCH = KEYS // CH
GR = CH // 16

CHT = 4096
NCT = MS // CHT

_LOG2E = 1.4426950408889634
_LN2 = 0.6931471805599453
_RND = 12582912.0


def _bf16_rne(x):
    i = lax.bitcast_convert_type(x, jnp.int32)
    lsb = lax.shift_right_logical(i, 16) & 1
    r = (i + 0x7FFF + lsb) & jnp.int32(-65536)
    return lax.bitcast_convert_type(r, jnp.float32)


def _exp_precise(x):
    x = jnp.maximum(x, -87.0)
    t = x * _LOG2E
    n_f = (t + _RND) - _RND
    r = (t - n_f) * _LN2
    p = 1.0 + r * (1.0 + r * (0.5 + r * (1.0 / 6.0 + r * (
        1.0 / 24.0 + r * (1.0 / 120.0 + r * (1.0 / 720.0))))))
    n_i = n_f.astype(jnp.int32)
    scale = lax.bitcast_convert_type((n_i + 127) << 23, jnp.float32)
    return p * scale


_mesh = plsc.VectorSubcoreMesh(core_axis_name="c", subcore_axis_name="s")


@functools.partial(
    pl.kernel,
    mesh=_mesh,
    compiler_params=pltpu.CompilerParams(needs_layout_passes=False),
    out_type=[
        jax.ShapeDtypeStruct((NW * 64,), jnp.float32),
        jax.ShapeDtypeStruct((NW * 32,), jnp.float32),
    ],
    scratch_types=[
        pltpu.VMEM((CH, D), jnp.float32),
        pltpu.VMEM((CH, D), jnp.float32),
        pltpu.VMEM((KEYS,), jnp.float32),
        pltpu.VMEM((KEYS,), jnp.int32),
        pltpu.VMEM((64,), jnp.float32),
        pltpu.VMEM((16,), jnp.float32),
        pltpu.VMEM((64,), jnp.float32),
        pltpu.VMEM((32,), jnp.float32),
        pltpu.SemaphoreType.DMA,
        pltpu.SemaphoreType.DMA,
    ],
)
def _sc_attn(q_hbm, k_hbm, v_hbm, pos_hbm, qp_hbm, acc_out, st_out,
             buf0, buf1, scores, posb, qv, qpb, stacc, ststat, sem0, sem1):
    cid = lax.axis_index("c")
    sid = lax.axis_index("s")
    wid = cid * 16 + sid
    b = wid // SEG
    seg = wid % SEG
    base = MS + seg * KEYS

    pltpu.sync_copy(q_hbm.at[b], qv)
    pltpu.sync_copy(qp_hbm, qpb)
    pltpu.sync_copy(pos_hbm.at[b, pl.ds(base, KEYS)], posb)

    lane = lax.iota(jnp.int32, 16)
    q0 = _bf16_rne(qv[pl.ds(0, 16)])
    q1 = _bf16_rne(qv[pl.ds(16, 16)])
    q2 = _bf16_rne(qv[pl.ds(32, 16)])
    q3 = _bf16_rne(qv[pl.ds(48, 16)])
    qp_vec = qpb[...]

    def kcopy(c, buf, sem):
        return pltpu.make_async_copy(
            k_hbm.at[b, pl.ds(base + c * CH, CH), :], buf, sem)

    def vcopy(c, buf, sem):
        return pltpu.make_async_copy(
            v_hbm.at[b, pl.ds(base + c * CH, CH), :], buf, sem)

    def p1_chunk(c, buf, m_run):
        def group(gi, m_run):
            grow = gi * 16
            svec = jnp.zeros((16,), jnp.float32)
            for j in range(16):
                row = grow + j
                p = (q0 * _bf16_rne(buf[row, pl.ds(0, 16)])
                     + q1 * _bf16_rne(buf[row, pl.ds(16, 16)])
                     + q2 * _bf16_rne(buf[row, pl.ds(32, 16)])
                     + q3 * _bf16_rne(buf[row, pl.ds(48, 16)]))
                svec = jnp.where(lane == j, jnp.sum(p), svec)
            gk = c * GR + gi
            pos16 = posb[pl.ds(gk * 16, 16)].astype(jnp.float32)
            svec = svec * SCALE - SLOPE * jnp.abs(qp_vec - pos16)
            scores[pl.ds(gk * 16, 16)] = svec
            return jnp.maximum(m_run, svec)
        return lax.fori_loop(0, GR, group, m_run)

    kcopy(0, buf0, sem0).start()

    def outer1(i, m_run):
        c0 = 2 * i
        kcopy(c0, buf0, sem0).wait()
        kcopy(c0 + 1, buf1, sem1).start()
        m_run = p1_chunk(c0, buf0, m_run)
        kcopy(c0 + 1, buf1, sem1).wait()

        @pl.when(i < NCH // 2 - 1)
        def _():
            kcopy(c0 + 2, buf0, sem0).start()

        m_run = p1_chunk(c0 + 1, buf1, m_run)
        return m_run

    m_run = lax.fori_loop(0, NCH // 2, outer1,
                          jnp.full((16,), -3e38, jnp.float32))
    gmax = jnp.max(m_run)

    def p2_chunk(c, buf, carry):
        def group(gi, carry):
            a0, a1, a2, a3, ws = carry
            gk = c * GR + gi
            svec = scores[pl.ds(gk * 16, 16)]
            w = _exp_precise(svec - gmax)
            ws = ws + w
            grow = gi * 16
            for j in range(16):
                row = grow + j
                wj = w[j]
                a0 = a0 + wj * buf[row, pl.ds(0, 16)]
                a1 = a1 + wj * buf[row, pl.ds(16, 16)]
                a2 = a2 + wj * buf[row, pl.ds(32, 16)]
                a3 = a3 + wj * buf[row, pl.ds(48, 16)]
            return (a0, a1, a2, a3, ws)
        return lax.fori_loop(0, GR, group, carry)

    vcopy(0, buf0, sem0).start()
    zero = jnp.zeros((16,), jnp.float32)

    def outer2(i, carry):
        c0 = 2 * i
        vcopy(c0, buf0, sem0).wait()
        vcopy(c0 + 1, buf1, sem1).start()
        carry = p2_chunk(c0, buf0, carry)
        vcopy(c0 + 1, buf1, sem1).wait()

        @pl.when(i < NCH // 2 - 1)
        def _():
            vcopy(c0 + 2, buf0, sem0).start()

        carry = p2_chunk(c0 + 1, buf1, carry)
        return carry

    a0, a1, a2, a3, ws = lax.fori_loop(0, NCH // 2, outer2,
                                       (zero, zero, zero, zero, zero))

    stacc[pl.ds(0, 16)] = a0
    stacc[pl.ds(16, 16)] = a1
    stacc[pl.ds(32, 16)] = a2
    stacc[pl.ds(48, 16)] = a3
    pltpu.sync_copy(stacc, acc_out.at[pl.ds(wid * 64, 64)])

    ststat[pl.ds(0, 16)] = zero + gmax
    ststat[pl.ds(16, 16)] = ws
    pltpu.sync_copy(ststat, st_out.at[pl.ds(wid * 32, 32)])


def _tc_body(q_ref, kt_ref, vt_ref, pos_ref, qp_ref, acc_out, st_out):
    qp = qp_ref[0]
    q16 = q_ref[0].astype(jnp.bfloat16)
    k16 = kt_ref[0].astype(jnp.bfloat16)
    s = lax.dot_general(q16, k16, (((1,), (0,)), ((), ())),
                        preferred_element_type=jnp.float32)
    pos = pos_ref[0].astype(jnp.float32)
    s = s * SCALE - SLOPE * jnp.abs(qp - pos)
    m_c = jnp.max(s)
    p = jnp.exp(s - m_c)
    l_c = jnp.sum(p)
    pv = lax.dot_general(p.astype(jnp.bfloat16),
                         vt_ref[0].astype(jnp.bfloat16),
                         (((1,), (1,)), ((), ())),
                         preferred_element_type=jnp.float32)
    acc_out[...] = pv.reshape(1, 1, 1, 64)
    iota = lax.broadcasted_iota(jnp.int32, (1, 1, 1, 128), 3)
    st_out[...] = jnp.where(iota == 0, m_c,
                            jnp.where(iota == 1, l_c, 0.0))


_tc_attn = pl.pallas_call(
    _tc_body,
    grid=(B, NCT if NCT else 1),
    in_specs=[
        pl.BlockSpec((1, 1, 64), lambda b, c: (b, 0, 0)),
        pl.BlockSpec((1, 64, CHT), lambda b, c: (b, 0, c)),
        pl.BlockSpec((1, 64, CHT), lambda b, c: (b, 0, c)),
        pl.BlockSpec((1, 1, CHT), lambda b, c: (b * (NCT if NCT else 1) + c, 0, 0)),
        pl.BlockSpec(memory_space=pltpu.MemorySpace.SMEM),
    ],
    out_specs=[
        pl.BlockSpec((1, 1, 1, 64), lambda b, c: (b, c, 0, 0)),
        pl.BlockSpec((1, 1, 1, 128), lambda b, c: (b, c, 0, 0)),
    ],
    out_shape=[
        jax.ShapeDtypeStruct((B, NCT if NCT else 1, 1, 64), jnp.float32),
        jax.ShapeDtypeStruct((B, NCT if NCT else 1, 1, 128), jnp.float32),
    ],
)


def kernel(query_addr, key_addrs, values, query_pos, key_pos):
    pos32 = key_pos.astype(jnp.int32)
    qp_s = jnp.full((1,), query_pos, dtype=jnp.float32)

    parts_m = []
    parts_l = []
    parts_acc = []

    if MSC:
        qp = jnp.full((16,), query_pos, dtype=jnp.float32)
        accf, stf = _sc_attn(query_addr, key_addrs, values, pos32, qp)
        acc = accf.reshape(B, SEG, 64)
        st = stf.reshape(B, SEG, 32)
        parts_m.append(st[:, :, 0])
        parts_l.append(st[:, :, 16:32].sum(-1))
        parts_acc.append(acc)

    if NCT:
        q3d = query_addr.reshape(B, 1, 64)
        pos_tc = pos32[:, :MS].reshape(B * NCT, 1, CHT)
        kt = jnp.swapaxes(key_addrs, 1, 2)
        vt = jnp.swapaxes(values, 1, 2)
        tacc, tst = _tc_attn(q3d, kt, vt, pos_tc, qp_s)
        parts_m.append(tst[:, :, 0, 0])
        parts_l.append(tst[:, :, 0, 1])
        parts_acc.append(tacc[:, :, 0, :])

    pm = jnp.concatenate(parts_m, axis=1)
    pl_ = jnp.concatenate(parts_l, axis=1)
    pa = jnp.concatenate(parts_acc, axis=1)
    gm = pm.max(axis=1, keepdims=True)
    sc = jnp.exp(pm - gm)
    num = (sc[:, :, None] * pa).sum(1)
    den = (sc * pl_).sum(1)[:, None]
    return num / den
